# baseline (speedup 1.0000x reference)
import functools

import jax
import jax.numpy as jnp
from jax import lax
from jax.experimental import pallas as pl
from jax.experimental.pallas import tpu as pltpu
from jax.experimental.pallas import tpu_sc as plsc

N = 10000
E = 320000
D = 128
BN_EPS = 1e-5

NC = 2
NS = 16
NW = NC * NS
B = 64
NBUF = 4
BLK = 32
NBLK = 5
CH = BLK * NBLK
EPAD = NW * CH * B
NPAD = 10112
SEG = NPAD // NS

_mesh = plsc.VectorSubcoreMesh(core_axis_name="c", subcore_axis_name="s")


@functools.partial(
    pl.kernel,
    out_type=jax.ShapeDtypeStruct((NC * NPAD,), jnp.float32),
    mesh=_mesh,
    scratch_types=[
        pltpu.VMEM_SHARED((NPAD,), jnp.float32),
        pltpu.VMEM((CH, B), jnp.int32),
        pltpu.VMEM((B,), jnp.float32),
        pltpu.VMEM((640,), jnp.float32),
    ],
)
def _sc_degree(dst_hbm, out_hbm, deg_sh, idx_d, ones_v, zeros_v):
    cid = lax.axis_index("c")
    sid = lax.axis_index("s")
    wid = cid * NS + sid

    def _fill_ones(i, _):
        ones_v[pl.ds(i * 16, 16)] = jnp.full((16,), 1.0, jnp.float32)
        return 0

    def _fill_zeros(i, _):
        zeros_v[pl.ds(i * 16, 16)] = jnp.zeros((16,), jnp.float32)
        return 0

    lax.fori_loop(0, B // 16, _fill_ones, 0)
    lax.fori_loop(0, 640 // 16, _fill_zeros, 0)

    pltpu.sync_copy(dst_hbm.at[wid], idx_d)
    pltpu.sync_copy(zeros_v.at[pl.ds(0, SEG)],
                    deg_sh.at[pl.ds(sid * SEG, SEG)])
    plsc.subcore_barrier()

    def _body(j, _):
        pltpu.sync_copy(ones_v, deg_sh.at[idx_d.at[j]], add=True)
        return 0

    lax.fori_loop(0, CH, _body, 0)
    plsc.subcore_barrier()
    pltpu.sync_copy(deg_sh.at[pl.ds(sid * SEG, SEG)],
                    zeros_v.at[pl.ds(0, SEG)])
    pltpu.sync_copy(zeros_v.at[pl.ds(0, SEG)],
                    out_hbm.at[pl.ds(cid * NPAD + sid * SEG, SEG)])


@functools.partial(
    pl.kernel,
    out_type=jax.ShapeDtypeStruct((NC, NPAD, D), jnp.float32),
    mesh=_mesh,
    scratch_types=[
        pltpu.VMEM_SHARED((NPAD, D), jnp.float32),
        pltpu.VMEM((BLK, B), jnp.int32),
        pltpu.VMEM((BLK, B), jnp.int32),
        pltpu.VMEM((B, D), jnp.float32),
        pltpu.VMEM((B, D), jnp.float32),
        pltpu.VMEM((B, D), jnp.float32),
        pltpu.VMEM((B, D), jnp.float32),
        pltpu.SemaphoreType.DMA,
        pltpu.SemaphoreType.DMA,
        pltpu.SemaphoreType.DMA,
        pltpu.SemaphoreType.DMA,
        pltpu.SemaphoreType.DMA,
        pltpu.SemaphoreType.DMA,
        pltpu.SemaphoreType.DMA,
        pltpu.SemaphoreType.DMA,
    ],
)
def _sc_msg(xws_hbm, src_hbm, dst_hbm, out_hbm, acc_sh, idx_s, idx_d,
            r0, r1, r2, r3, g0, g1, g2, g3, s0, s1, s2, s3):
    cid = lax.axis_index("c")
    sid = lax.axis_index("s")
    wid = cid * NS + sid
    rows = (r0, r1, r2, r3)
    gsem = (g0, g1, g2, g3)
    ssem = (s0, s1, s2, s3)

    def _zrow(i, _):
        def _z16(j, _):
            r0[i, pl.ds(j * 16, 16)] = jnp.zeros((16,), jnp.float32)
            return 0
        lax.fori_loop(0, D // 16, _z16, 0)
        return 0

    lax.fori_loop(0, B, _zrow, 0)

    def _zcpy(k, _):
        pltpu.sync_copy(r0, acc_sh.at[pl.ds(sid * SEG + k * B, B), :])
        return 0

    lax.fori_loop(0, SEG // B, _zcpy, 0)
    pltpu.sync_copy(r0.at[pl.ds(0, SEG - (SEG // B) * B), :],
                    acc_sh.at[pl.ds(sid * SEG + (SEG // B) * B,
                                    SEG - (SEG // B) * B), :])
    plsc.subcore_barrier()

    def _block(b, _):
        pltpu.sync_copy(src_hbm.at[wid, pl.ds(b * BLK, BLK)], idx_s)
        pltpu.sync_copy(dst_hbm.at[wid, pl.ds(b * BLK, BLK)], idx_d)
        for p in range(NBUF):
            pltpu.async_copy(xws_hbm.at[idx_s.at[p]], rows[p], gsem[p])

        def _grp(u, _):
            j0 = NBUF * u
            for p in range(NBUF):
                j = j0 + p
                q = (p + NBUF - 1) % NBUF
                pltpu.make_async_copy(
                    xws_hbm.at[idx_s.at[j]], rows[p], gsem[p]).wait()
                pltpu.async_copy(rows[p], acc_sh.at[idx_d.at[j]], ssem[p],
                                 add=True)

                @pl.when((j >= 1) & (j + NBUF - 1 < BLK))
                def _(j=j, q=q):
                    pltpu.make_async_copy(
                        rows[q], acc_sh.at[idx_d.at[j - 1]], ssem[q]).wait()
                    pltpu.async_copy(
                        xws_hbm.at[idx_s.at[j + NBUF - 1]], rows[q], gsem[q])
            return 0

        lax.fori_loop(0, BLK // NBUF, _grp, 0)
        for i in range(NBUF):
            pltpu.make_async_copy(
                rows[i], acc_sh.at[idx_d.at[BLK - NBUF + i]],
                ssem[i]).wait()
        return 0

    lax.fori_loop(0, NBLK, _block, 0)
    plsc.subcore_barrier()

    pltpu.sync_copy(acc_sh.at[pl.ds(sid * SEG, SEG), :],
                    out_hbm.at[cid, pl.ds(sid * SEG, SEG), :])


def _tc_first_body(x_ref, w_ref, deg_ref, out_ref):
    dis = lax.rsqrt(deg_ref[...])
    xw = jnp.dot(x_ref[...], w_ref[...], preferred_element_type=jnp.float32)
    out_ref[...] = xw * dis


def _tc_mid_body(acc_ref, xws_ref, deg_ref, w_ref, b_ref, g_ref, be_ref,
                 out_ref):
    dis = lax.rsqrt(deg_ref[...])
    acc = acc_ref[0] + acc_ref[1]
    conv = (acc + xws_ref[...]) * dis + b_ref[...]
    gs = g_ref[...] * lax.rsqrt(jnp.float32(1.0 + BN_EPS))
    h = jnp.maximum(conv * gs + be_ref[...], 0.0)
    xw = jnp.dot(h, w_ref[...], preferred_element_type=jnp.float32)
    out_ref[...] = xw * dis


def _tc_last_body(acc_ref, xws_ref, deg_ref, b_ref, g_ref, be_ref, out_ref):
    dis = lax.rsqrt(deg_ref[...])
    acc = acc_ref[0] + acc_ref[1]
    conv = (acc + xws_ref[...]) * dis + b_ref[...]
    gs = g_ref[...] * lax.rsqrt(jnp.float32(1.0 + BN_EPS))
    out_ref[...] = jnp.maximum(conv * gs + be_ref[...], 0.0)


BR = 2000
_row = pl.BlockSpec((BR, D), lambda i: (i, 0))
_deg_bs = pl.BlockSpec((BR, 1), lambda i: (i, 0))
_acc_bs = pl.BlockSpec((2, BR, D), lambda i: (0, i, 0))
_w_bs = pl.BlockSpec((D, D), lambda i: (0, 0))
_vec_bs = pl.BlockSpec((1, D), lambda i: (0, 0))


def _tc_first(x, w, deg):
    return pl.pallas_call(
        _tc_first_body,
        grid=(N // BR,),
        in_specs=[_row, _w_bs, _deg_bs],
        out_specs=_row,
        out_shape=jax.ShapeDtypeStruct((N, D), jnp.float32),
    )(x, w, deg)


def _tc_mid(acc, xws, deg, w, b, g, be):
    return pl.pallas_call(
        _tc_mid_body,
        grid=(N // BR,),
        in_specs=[_acc_bs, _row, _deg_bs, _w_bs, _vec_bs, _vec_bs, _vec_bs],
        out_specs=_row,
        out_shape=jax.ShapeDtypeStruct((N, D), jnp.float32),
    )(acc, xws, deg, w, b, g, be)


def _tc_last(acc, xws, deg, b, g, be):
    return pl.pallas_call(
        _tc_last_body,
        grid=(N // BR,),
        in_specs=[_acc_bs, _row, _deg_bs, _vec_bs, _vec_bs, _vec_bs],
        out_specs=_row,
        out_shape=jax.ShapeDtypeStruct((N, D), jnp.float32),
    )(acc, xws, deg, b, g, be)


@jax.jit
def kernel(x, edge_index, W1, b1, g1, be1, W2, b2, g2, be2, Wf, bf, gf, bef):
    src = edge_index[0]
    dst = edge_index[1]
    pad = EPAD - E
    ar = jnp.arange(pad, dtype=jnp.int32)
    src_p = jnp.concatenate([src, (ar * 37) % N]).reshape(NW, CH, B)
    dst_p = jnp.concatenate([dst, N + (ar % 96)]).reshape(NW, CH, B)

    degp = _sc_degree(dst_p)
    deg = (degp[:N] + degp[NPAD:NPAD + N] + 1.0)[:, None]

    b1r, g1r, be1r = b1[None, :], g1[None, :], be1[None, :]
    b2r, g2r, be2r = b2[None, :], g2[None, :], be2[None, :]
    bfr, gfr, befr = bf[None, :], gf[None, :], bef[None, :]

    xws1 = _tc_first(x, W1, deg)
    acc1 = _sc_msg(xws1, src_p, dst_p)
    xws2 = _tc_mid(acc1, xws1, deg, W2, b1r, g1r, be1r)
    acc2 = _sc_msg(xws2, src_p, dst_p)
    xws3 = _tc_mid(acc2, xws2, deg, Wf, b2r, g2r, be2r)
    acc3 = _sc_msg(xws3, src_p, dst_p)
    return _tc_last(acc3, xws3, deg, bfr, gfr, befr)


# speedup vs baseline: 1.0094x; 1.0094x over previous
import functools

import jax
import jax.numpy as jnp
from jax import lax
from jax.experimental import pallas as pl
from jax.experimental.pallas import tpu as pltpu
from jax.experimental.pallas import tpu_sc as plsc

N = 10000
E = 320000
D = 128
BN_EPS = 1e-5

NC = 2
NS = 16
NW = NC * NS
B = 64
NBUF = 4
BLK = 16
NBLK = 10
CH = BLK * NBLK
EPAD = NW * CH * B
NPAD = 10112
SEG = NPAD // NS

_mesh = plsc.VectorSubcoreMesh(core_axis_name="c", subcore_axis_name="s")


@functools.partial(
    pl.kernel,
    out_type=jax.ShapeDtypeStruct((NC * NPAD,), jnp.float32),
    mesh=_mesh,
    scratch_types=[
        pltpu.VMEM_SHARED((NPAD,), jnp.float32),
        pltpu.VMEM((CH, B), jnp.int32),
        pltpu.VMEM((B,), jnp.float32),
        pltpu.VMEM((640,), jnp.float32),
        pltpu.SemaphoreType.DMA,
    ],
)
def _sc_degree(dst_hbm, out_hbm, deg_sh, idx_d, ones_v, zeros_v, sem_d):
    cid = lax.axis_index("c")
    sid = lax.axis_index("s")
    wid = cid * NS + sid

    def _fill_ones(i, _):
        ones_v[pl.ds(i * 16, 16)] = jnp.full((16,), 1.0, jnp.float32)
        return 0

    def _fill_zeros(i, _):
        zeros_v[pl.ds(i * 16, 16)] = jnp.zeros((16,), jnp.float32)
        return 0

    lax.fori_loop(0, B // 16, _fill_ones, 0)
    lax.fori_loop(0, 640 // 16, _fill_zeros, 0)

    pltpu.sync_copy(dst_hbm.at[wid], idx_d)
    pltpu.sync_copy(zeros_v.at[pl.ds(0, SEG)],
                    deg_sh.at[pl.ds(sid * SEG, SEG)])
    plsc.subcore_barrier()

    def _fire(j, _):
        pltpu.async_copy(ones_v, deg_sh.at[idx_d.at[j]], sem_d, add=True)
        return 0

    def _drain(j, _):
        pltpu.make_async_copy(ones_v, deg_sh.at[idx_d.at[j]], sem_d).wait()
        return 0

    lax.fori_loop(0, CH, _fire, 0)
    lax.fori_loop(0, CH, _drain, 0)
    plsc.subcore_barrier()
    pltpu.sync_copy(deg_sh.at[pl.ds(sid * SEG, SEG)],
                    zeros_v.at[pl.ds(0, SEG)])
    pltpu.sync_copy(zeros_v.at[pl.ds(0, SEG)],
                    out_hbm.at[pl.ds(cid * NPAD + sid * SEG, SEG)])


@functools.partial(
    pl.kernel,
    out_type=jax.ShapeDtypeStruct((NC, NPAD, D), jnp.float32),
    mesh=_mesh,
    scratch_types=[
        pltpu.VMEM_SHARED((NPAD, D), jnp.float32),
        pltpu.VMEM((BLK, B), jnp.int32),
        pltpu.VMEM((BLK, B), jnp.int32),
        pltpu.VMEM((BLK, B), jnp.int32),
        pltpu.VMEM((BLK, B), jnp.int32),
        pltpu.VMEM((B, D), jnp.float32),
        pltpu.VMEM((B, D), jnp.float32),
        pltpu.VMEM((B, D), jnp.float32),
        pltpu.VMEM((B, D), jnp.float32),
        pltpu.SemaphoreType.DMA,
        pltpu.SemaphoreType.DMA,
        pltpu.SemaphoreType.DMA,
        pltpu.SemaphoreType.DMA,
        pltpu.SemaphoreType.DMA,
        pltpu.SemaphoreType.DMA,
        pltpu.SemaphoreType.DMA,
        pltpu.SemaphoreType.DMA,
        pltpu.SemaphoreType.DMA,
        pltpu.SemaphoreType.DMA,
        pltpu.SemaphoreType.DMA,
        pltpu.SemaphoreType.DMA,
    ],
)
def _sc_msg(xws_hbm, src_hbm, dst_hbm, out_hbm, acc_sh, is0, id0, is1, id1,
            r0, r1, r2, r3, g0, g1, g2, g3, s0, s1, s2, s3,
            i0, i1, i2, i3):
    cid = lax.axis_index("c")
    sid = lax.axis_index("s")
    wid = cid * NS + sid
    rows = (r0, r1, r2, r3)
    gsem = (g0, g1, g2, g3)
    ssem = (s0, s1, s2, s3)

    def _zrow(i, _):
        def _z16(j, _):
            r0[i, pl.ds(j * 16, 16)] = jnp.zeros((16,), jnp.float32)
            return 0
        lax.fori_loop(0, D // 16, _z16, 0)
        return 0

    lax.fori_loop(0, B, _zrow, 0)

    def _zcpy(k, _):
        pltpu.sync_copy(r0, acc_sh.at[pl.ds(sid * SEG + k * B, B), :])
        return 0

    lax.fori_loop(0, SEG // B, _zcpy, 0)
    pltpu.sync_copy(r0.at[pl.ds(0, SEG - (SEG // B) * B), :],
                    acc_sh.at[pl.ds(sid * SEG + (SEG // B) * B,
                                    SEG - (SEG // B) * B), :])
    plsc.subcore_barrier()

    pltpu.async_copy(src_hbm.at[wid, pl.ds(0, BLK)], is0, i0)
    pltpu.async_copy(dst_hbm.at[wid, pl.ds(0, BLK)], id0, i1)
    pltpu.async_copy(src_hbm.at[wid, pl.ds(BLK, BLK)], is1, i2)
    pltpu.async_copy(dst_hbm.at[wid, pl.ds(BLK, BLK)], id1, i3)

    def _run_block(b, idx_s, idx_d, sem_is, sem_id):
        pltpu.make_async_copy(
            src_hbm.at[wid, pl.ds(b * BLK, BLK)], idx_s, sem_is).wait()
        pltpu.make_async_copy(
            dst_hbm.at[wid, pl.ds(b * BLK, BLK)], idx_d, sem_id).wait()
        for p in range(NBUF):
            pltpu.async_copy(xws_hbm.at[idx_s.at[p]], rows[p], gsem[p])

        def _grp(u, _):
            j0 = NBUF * u
            for p in range(NBUF):
                j = j0 + p
                q = (p + NBUF - 1) % NBUF
                pltpu.make_async_copy(
                    xws_hbm.at[idx_s.at[j]], rows[p], gsem[p]).wait()
                pltpu.async_copy(rows[p], acc_sh.at[idx_d.at[j]], ssem[p],
                                 add=True)

                @pl.when((j >= 1) & (j + NBUF - 1 < BLK))
                def _(j=j, q=q):
                    pltpu.make_async_copy(
                        rows[q], acc_sh.at[idx_d.at[j - 1]], ssem[q]).wait()
                    pltpu.async_copy(
                        xws_hbm.at[idx_s.at[j + NBUF - 1]], rows[q], gsem[q])
            return 0

        lax.fori_loop(0, BLK // NBUF, _grp, 0)
        for i in range(NBUF):
            pltpu.make_async_copy(
                rows[i], acc_sh.at[idx_d.at[BLK - NBUF + i]],
                ssem[i]).wait()

        @pl.when(b + 2 < NBLK)
        def _():
            pltpu.async_copy(
                src_hbm.at[wid, pl.ds((b + 2) * BLK, BLK)], idx_s, sem_is)
            pltpu.async_copy(
                dst_hbm.at[wid, pl.ds((b + 2) * BLK, BLK)], idx_d, sem_id)

    def _bpair(v, _):
        _run_block(2 * v, is0, id0, i0, i1)
        _run_block(2 * v + 1, is1, id1, i2, i3)
        return 0

    lax.fori_loop(0, NBLK // 2, _bpair, 0)
    plsc.subcore_barrier()

    pltpu.sync_copy(acc_sh.at[pl.ds(sid * SEG, SEG), :],
                    out_hbm.at[cid, pl.ds(sid * SEG, SEG), :])


def _tc_first_body(x_ref, w_ref, deg_ref, out_ref):
    dis = lax.rsqrt(deg_ref[...])
    xw = jnp.dot(x_ref[...], w_ref[...], preferred_element_type=jnp.float32)
    out_ref[...] = xw * dis


def _tc_mid_body(acc_ref, xws_ref, deg_ref, w_ref, b_ref, g_ref, be_ref,
                 out_ref):
    dis = lax.rsqrt(deg_ref[...])
    acc = acc_ref[0] + acc_ref[1]
    conv = (acc + xws_ref[...]) * dis + b_ref[...]
    gs = g_ref[...] * lax.rsqrt(jnp.float32(1.0 + BN_EPS))
    h = jnp.maximum(conv * gs + be_ref[...], 0.0)
    xw = jnp.dot(h, w_ref[...], preferred_element_type=jnp.float32)
    out_ref[...] = xw * dis


def _tc_last_body(acc_ref, xws_ref, deg_ref, b_ref, g_ref, be_ref, out_ref):
    dis = lax.rsqrt(deg_ref[...])
    acc = acc_ref[0] + acc_ref[1]
    conv = (acc + xws_ref[...]) * dis + b_ref[...]
    gs = g_ref[...] * lax.rsqrt(jnp.float32(1.0 + BN_EPS))
    out_ref[...] = jnp.maximum(conv * gs + be_ref[...], 0.0)


BR = 2000
_row = pl.BlockSpec((BR, D), lambda i: (i, 0))
_deg_bs = pl.BlockSpec((BR, 1), lambda i: (i, 0))
_acc_bs = pl.BlockSpec((2, BR, D), lambda i: (0, i, 0))
_w_bs = pl.BlockSpec((D, D), lambda i: (0, 0))
_vec_bs = pl.BlockSpec((1, D), lambda i: (0, 0))


def _tc_first(x, w, deg):
    return pl.pallas_call(
        _tc_first_body,
        grid=(N // BR,),
        in_specs=[_row, _w_bs, _deg_bs],
        out_specs=_row,
        out_shape=jax.ShapeDtypeStruct((N, D), jnp.float32),
    )(x, w, deg)


def _tc_mid(acc, xws, deg, w, b, g, be):
    return pl.pallas_call(
        _tc_mid_body,
        grid=(N // BR,),
        in_specs=[_acc_bs, _row, _deg_bs, _w_bs, _vec_bs, _vec_bs, _vec_bs],
        out_specs=_row,
        out_shape=jax.ShapeDtypeStruct((N, D), jnp.float32),
    )(acc, xws, deg, w, b, g, be)


def _tc_last(acc, xws, deg, b, g, be):
    return pl.pallas_call(
        _tc_last_body,
        grid=(N // BR,),
        in_specs=[_acc_bs, _row, _deg_bs, _vec_bs, _vec_bs, _vec_bs],
        out_specs=_row,
        out_shape=jax.ShapeDtypeStruct((N, D), jnp.float32),
    )(acc, xws, deg, b, g, be)


@jax.jit
def kernel(x, edge_index, W1, b1, g1, be1, W2, b2, g2, be2, Wf, bf, gf, bef):
    src = edge_index[0]
    dst = edge_index[1]
    pad = EPAD - E
    ar = jnp.arange(pad, dtype=jnp.int32)
    src_p = jnp.concatenate([src, (ar * 37) % N]).reshape(NW, CH, B)
    dst_p = jnp.concatenate([dst, N + (ar % 96)]).reshape(NW, CH, B)

    degp = _sc_degree(dst_p)
    deg = (degp[:N] + degp[NPAD:NPAD + N] + 1.0)[:, None]

    b1r, g1r, be1r = b1[None, :], g1[None, :], be1[None, :]
    b2r, g2r, be2r = b2[None, :], g2[None, :], be2[None, :]
    bfr, gfr, befr = bf[None, :], gf[None, :], bef[None, :]

    xws1 = _tc_first(x, W1, deg)
    acc1 = _sc_msg(xws1, src_p, dst_p)
    xws2 = _tc_mid(acc1, xws1, deg, W2, b1r, g1r, be1r)
    acc2 = _sc_msg(xws2, src_p, dst_p)
    xws3 = _tc_mid(acc2, xws2, deg, Wf, b2r, g2r, be2r)
    acc3 = _sc_msg(xws3, src_p, dst_p)
    return _tc_last(acc3, xws3, deg, bfr, gfr, befr)


# speedup vs baseline: 1.0578x; 1.0480x over previous
import functools

import jax
import jax.numpy as jnp
from jax import lax
from jax.experimental import pallas as pl
from jax.experimental.pallas import tpu as pltpu
from jax.experimental.pallas import tpu_sc as plsc

N = 10000
E = 320000
D = 128
BN_EPS = 1e-5

NC = 2
NS = 16
NW = NC * NS
B = 64
NBUF = 4
BLK = 32
NBLK = 5
CH = BLK * NBLK
EPAD = NW * CH * B
NPAD = 10112
SEG = NPAD // NS

_mesh = plsc.VectorSubcoreMesh(core_axis_name="c", subcore_axis_name="s")


@functools.partial(
    pl.kernel,
    out_type=jax.ShapeDtypeStruct((NC * NPAD,), jnp.float32),
    mesh=_mesh,
    scratch_types=[
        pltpu.VMEM_SHARED((NPAD,), jnp.float32),
        pltpu.VMEM((CH, B), jnp.int32),
        pltpu.VMEM((B,), jnp.float32),
        pltpu.VMEM((640,), jnp.float32),
        pltpu.SemaphoreType.DMA,
    ],
)
def _sc_degree(dst_hbm, out_hbm, deg_sh, idx_d, ones_v, zeros_v, sem_d):
    cid = lax.axis_index("c")
    sid = lax.axis_index("s")
    wid = cid * NS + sid

    def _fill_ones(i, _):
        ones_v[pl.ds(i * 16, 16)] = jnp.full((16,), 1.0, jnp.float32)
        return 0

    def _fill_zeros(i, _):
        zeros_v[pl.ds(i * 16, 16)] = jnp.zeros((16,), jnp.float32)
        return 0

    lax.fori_loop(0, B // 16, _fill_ones, 0)
    lax.fori_loop(0, 640 // 16, _fill_zeros, 0)

    pltpu.sync_copy(dst_hbm.at[wid], idx_d)
    pltpu.sync_copy(zeros_v.at[pl.ds(0, SEG)],
                    deg_sh.at[pl.ds(sid * SEG, SEG)])
    plsc.subcore_barrier()

    def _fire(j, _):
        pltpu.async_copy(ones_v, deg_sh.at[idx_d.at[j]], sem_d, add=True)
        return 0

    def _drain(j, _):
        pltpu.make_async_copy(ones_v, deg_sh.at[idx_d.at[j]], sem_d).wait()
        return 0

    lax.fori_loop(0, CH, _fire, 0)
    lax.fori_loop(0, CH, _drain, 0)
    plsc.subcore_barrier()
    pltpu.sync_copy(deg_sh.at[pl.ds(sid * SEG, SEG)],
                    zeros_v.at[pl.ds(0, SEG)])
    pltpu.sync_copy(zeros_v.at[pl.ds(0, SEG)],
                    out_hbm.at[pl.ds(cid * NPAD + sid * SEG, SEG)])


@functools.partial(
    pl.kernel,
    out_type=jax.ShapeDtypeStruct((NC, NPAD, D), jnp.float32),
    mesh=_mesh,
    scratch_types=[
        pltpu.VMEM_SHARED((NPAD, D), jnp.float32),
        pltpu.VMEM((BLK, B), jnp.int32),
        pltpu.VMEM((BLK, B), jnp.int32),
        pltpu.VMEM((BLK, B), jnp.int32),
        pltpu.VMEM((BLK, B), jnp.int32),
        pltpu.VMEM((B, D), jnp.float32),
        pltpu.VMEM((B, D), jnp.float32),
        pltpu.VMEM((B, D), jnp.float32),
        pltpu.VMEM((B, D), jnp.float32),
        pltpu.SemaphoreType.DMA,
        pltpu.SemaphoreType.DMA,
        pltpu.SemaphoreType.DMA,
        pltpu.SemaphoreType.DMA,
        pltpu.SemaphoreType.DMA,
        pltpu.SemaphoreType.DMA,
        pltpu.SemaphoreType.DMA,
        pltpu.SemaphoreType.DMA,
        pltpu.SemaphoreType.DMA,
        pltpu.SemaphoreType.DMA,
        pltpu.SemaphoreType.DMA,
        pltpu.SemaphoreType.DMA,
    ],
)
def _sc_msg(xws_hbm, src_hbm, dst_hbm, out_hbm, acc_sh, is0, id0, is1, id1,
            r0, r1, r2, r3, g0, g1, g2, g3, s0, s1, s2, s3,
            i0, i1, i2, i3):
    cid = lax.axis_index("c")
    sid = lax.axis_index("s")
    wid = cid * NS + sid
    rows = (r0, r1, r2, r3)
    gsem = (g0, g1, g2, g3)
    ssem = (s0, s1, s2, s3)

    def _zrow(i, _):
        def _z16(j, _):
            r0[i, pl.ds(j * 16, 16)] = jnp.zeros((16,), jnp.float32)
            return 0
        lax.fori_loop(0, D // 16, _z16, 0)
        return 0

    lax.fori_loop(0, B, _zrow, 0)

    def _zcpy(k, _):
        pltpu.sync_copy(r0, acc_sh.at[pl.ds(sid * SEG + k * B, B), :])
        return 0

    lax.fori_loop(0, SEG // B, _zcpy, 0)
    pltpu.sync_copy(r0.at[pl.ds(0, SEG - (SEG // B) * B), :],
                    acc_sh.at[pl.ds(sid * SEG + (SEG // B) * B,
                                    SEG - (SEG // B) * B), :])
    plsc.subcore_barrier()

    pltpu.async_copy(src_hbm.at[wid, pl.ds(0, BLK)], is0, i0)
    pltpu.async_copy(dst_hbm.at[wid, pl.ds(0, BLK)], id0, i1)
    pltpu.async_copy(src_hbm.at[wid, pl.ds(BLK, BLK)], is1, i2)
    pltpu.async_copy(dst_hbm.at[wid, pl.ds(BLK, BLK)], id1, i3)

    def _run_block(b, idx_s, idx_d, sem_is, sem_id):
        pltpu.make_async_copy(
            src_hbm.at[wid, pl.ds(b * BLK, BLK)], idx_s, sem_is).wait()
        pltpu.make_async_copy(
            dst_hbm.at[wid, pl.ds(b * BLK, BLK)], idx_d, sem_id).wait()
        for p in range(NBUF):
            pltpu.async_copy(xws_hbm.at[idx_s.at[p]], rows[p], gsem[p])

        def _grp(u, _):
            j0 = NBUF * u
            for p in range(NBUF):
                j = j0 + p
                q = (p + NBUF - 1) % NBUF
                pltpu.make_async_copy(
                    xws_hbm.at[idx_s.at[j]], rows[p], gsem[p]).wait()
                pltpu.async_copy(rows[p], acc_sh.at[idx_d.at[j]], ssem[p],
                                 add=True)

                @pl.when((j >= 1) & (j + NBUF - 1 < BLK))
                def _(j=j, q=q):
                    pltpu.make_async_copy(
                        rows[q], acc_sh.at[idx_d.at[j - 1]], ssem[q]).wait()
                    pltpu.async_copy(
                        xws_hbm.at[idx_s.at[j + NBUF - 1]], rows[q], gsem[q])
            return 0

        lax.fori_loop(0, BLK // NBUF, _grp, 0)
        for i in range(NBUF):
            pltpu.make_async_copy(
                rows[i], acc_sh.at[idx_d.at[BLK - NBUF + i]],
                ssem[i]).wait()

        @pl.when(b + 2 < NBLK)
        def _():
            pltpu.async_copy(
                src_hbm.at[wid, pl.ds((b + 2) * BLK, BLK)], idx_s, sem_is)
            pltpu.async_copy(
                dst_hbm.at[wid, pl.ds((b + 2) * BLK, BLK)], idx_d, sem_id)

    def _bpair(v, _):
        _run_block(2 * v, is0, id0, i0, i1)
        _run_block(2 * v + 1, is1, id1, i2, i3)
        return 0

    lax.fori_loop(0, NBLK // 2, _bpair, 0)
    if NBLK % 2:
        _run_block(NBLK - 1, is0, id0, i0, i1)
    plsc.subcore_barrier()

    pltpu.sync_copy(acc_sh.at[pl.ds(sid * SEG, SEG), :],
                    out_hbm.at[cid, pl.ds(sid * SEG, SEG), :])


def _tc_first_body(x_ref, w_ref, deg_ref, out_ref):
    dis = lax.rsqrt(deg_ref[...])
    xw = jnp.dot(x_ref[...], w_ref[...], preferred_element_type=jnp.float32)
    out_ref[...] = xw * dis


def _tc_mid_body(acc_ref, xws_ref, deg_ref, w_ref, b_ref, g_ref, be_ref,
                 out_ref):
    dis = lax.rsqrt(deg_ref[...])
    acc = acc_ref[0] + acc_ref[1]
    conv = (acc + xws_ref[...]) * dis + b_ref[...]
    gs = g_ref[...] * lax.rsqrt(jnp.float32(1.0 + BN_EPS))
    h = jnp.maximum(conv * gs + be_ref[...], 0.0)
    xw = jnp.dot(h, w_ref[...], preferred_element_type=jnp.float32)
    out_ref[...] = xw * dis


def _tc_last_body(acc_ref, xws_ref, deg_ref, b_ref, g_ref, be_ref, out_ref):
    dis = lax.rsqrt(deg_ref[...])
    acc = acc_ref[0] + acc_ref[1]
    conv = (acc + xws_ref[...]) * dis + b_ref[...]
    gs = g_ref[...] * lax.rsqrt(jnp.float32(1.0 + BN_EPS))
    out_ref[...] = jnp.maximum(conv * gs + be_ref[...], 0.0)


BR = 2000
_row = pl.BlockSpec((BR, D), lambda i: (i, 0))
_deg_bs = pl.BlockSpec((BR, 1), lambda i: (i, 0))
_acc_bs = pl.BlockSpec((2, BR, D), lambda i: (0, i, 0))
_w_bs = pl.BlockSpec((D, D), lambda i: (0, 0))
_vec_bs = pl.BlockSpec((1, D), lambda i: (0, 0))


def _tc_first(x, w, deg):
    return pl.pallas_call(
        _tc_first_body,
        grid=(N // BR,),
        in_specs=[_row, _w_bs, _deg_bs],
        out_specs=_row,
        out_shape=jax.ShapeDtypeStruct((N, D), jnp.float32),
    )(x, w, deg)


def _tc_mid(acc, xws, deg, w, b, g, be):
    return pl.pallas_call(
        _tc_mid_body,
        grid=(N // BR,),
        in_specs=[_acc_bs, _row, _deg_bs, _w_bs, _vec_bs, _vec_bs, _vec_bs],
        out_specs=_row,
        out_shape=jax.ShapeDtypeStruct((N, D), jnp.float32),
    )(acc, xws, deg, w, b, g, be)


def _tc_last(acc, xws, deg, b, g, be):
    return pl.pallas_call(
        _tc_last_body,
        grid=(N // BR,),
        in_specs=[_acc_bs, _row, _deg_bs, _vec_bs, _vec_bs, _vec_bs],
        out_specs=_row,
        out_shape=jax.ShapeDtypeStruct((N, D), jnp.float32),
    )(acc, xws, deg, b, g, be)


@jax.jit
def kernel(x, edge_index, W1, b1, g1, be1, W2, b2, g2, be2, Wf, bf, gf, bef):
    src = edge_index[0]
    dst = edge_index[1]
    pad = EPAD - E
    ar = jnp.arange(pad, dtype=jnp.int32)
    src_p = jnp.concatenate([src, (ar * 37) % N]).reshape(NW, CH, B)
    dst_p = jnp.concatenate([dst, N + (ar % 96)]).reshape(NW, CH, B)

    degp = _sc_degree(dst_p)
    deg = (degp[:N] + degp[NPAD:NPAD + N] + 1.0)[:, None]

    b1r, g1r, be1r = b1[None, :], g1[None, :], be1[None, :]
    b2r, g2r, be2r = b2[None, :], g2[None, :], be2[None, :]
    bfr, gfr, befr = bf[None, :], gf[None, :], bef[None, :]

    xws1 = _tc_first(x, W1, deg)
    acc1 = _sc_msg(xws1, src_p, dst_p)
    xws2 = _tc_mid(acc1, xws1, deg, W2, b1r, g1r, be1r)
    acc2 = _sc_msg(xws2, src_p, dst_p)
    xws3 = _tc_mid(acc2, xws2, deg, Wf, b2r, g2r, be2r)
    acc3 = _sc_msg(xws3, src_p, dst_p)
    return _tc_last(acc3, xws3, deg, bfr, gfr, befr)


# speedup vs baseline: 1.0756x; 1.0168x over previous
import functools

import jax
import jax.numpy as jnp
from jax import lax
from jax.experimental import pallas as pl
from jax.experimental.pallas import tpu as pltpu
from jax.experimental.pallas import tpu_sc as plsc

N = 10000
E = 320000
D = 128
BN_EPS = 1e-5

NC = 2
NS = 16
NW = NC * NS
B = 64
NBUF = 4
BLK = 32
NBLK = 5
CH = BLK * NBLK
EPAD = NW * CH * B
NPAD = 10112
SEG = NPAD // NS

_mesh = plsc.VectorSubcoreMesh(core_axis_name="c", subcore_axis_name="s")


@functools.partial(
    pl.kernel,
    out_type=jax.ShapeDtypeStruct((NC * NPAD,), jnp.float32),
    mesh=_mesh,
    scratch_types=[
        pltpu.VMEM_SHARED((NPAD,), jnp.float32),
        pltpu.VMEM((CH, B), jnp.int32),
        pltpu.VMEM((B,), jnp.float32),
        pltpu.VMEM((640,), jnp.float32),
        pltpu.SemaphoreType.DMA,
    ],
)
def _sc_degree(dst_hbm, out_hbm, deg_sh, idx_d, ones_v, zeros_v, sem_d):
    cid = lax.axis_index("c")
    sid = lax.axis_index("s")
    wid = cid * NS + sid

    def _fill_ones(i, _):
        ones_v[pl.ds(i * 16, 16)] = jnp.full((16,), 1.0, jnp.float32)
        return 0

    def _fill_zeros(i, _):
        zeros_v[pl.ds(i * 16, 16)] = jnp.zeros((16,), jnp.float32)
        return 0

    lax.fori_loop(0, B // 16, _fill_ones, 0)
    lax.fori_loop(0, 640 // 16, _fill_zeros, 0)

    pltpu.sync_copy(dst_hbm.at[wid], idx_d)
    pltpu.sync_copy(zeros_v.at[pl.ds(0, SEG)],
                    deg_sh.at[pl.ds(sid * SEG, SEG)])
    plsc.subcore_barrier()

    def _fire(j, _):
        pltpu.async_copy(ones_v, deg_sh.at[idx_d.at[j]], sem_d, add=True)
        return 0

    def _drain(j, _):
        pltpu.make_async_copy(ones_v, deg_sh.at[idx_d.at[j]], sem_d).wait()
        return 0

    lax.fori_loop(0, CH, _fire, 0)
    lax.fori_loop(0, CH, _drain, 0)
    plsc.subcore_barrier()
    pltpu.sync_copy(deg_sh.at[pl.ds(sid * SEG, SEG)],
                    zeros_v.at[pl.ds(0, SEG)])
    pltpu.sync_copy(zeros_v.at[pl.ds(0, SEG)],
                    out_hbm.at[pl.ds(cid * NPAD + sid * SEG, SEG)])


@functools.partial(
    pl.kernel,
    out_type=jax.ShapeDtypeStruct((NC, NPAD, D), jnp.float32),
    mesh=_mesh,
    scratch_types=[
        pltpu.VMEM_SHARED((NPAD, D), jnp.float32),
        pltpu.VMEM((BLK, B), jnp.int32),
        pltpu.VMEM((BLK, B), jnp.int32),
        pltpu.VMEM((BLK, B), jnp.int32),
        pltpu.VMEM((BLK, B), jnp.int32),
        pltpu.VMEM((B, D), jnp.float32),
        pltpu.VMEM((B, D), jnp.float32),
        pltpu.VMEM((B, D), jnp.float32),
        pltpu.VMEM((B, D), jnp.float32),
        pltpu.SemaphoreType.DMA,
        pltpu.SemaphoreType.DMA,
        pltpu.SemaphoreType.DMA,
        pltpu.SemaphoreType.DMA,
        pltpu.SemaphoreType.DMA,
        pltpu.SemaphoreType.DMA,
        pltpu.SemaphoreType.DMA,
        pltpu.SemaphoreType.DMA,
        pltpu.SemaphoreType.DMA,
        pltpu.SemaphoreType.DMA,
        pltpu.SemaphoreType.DMA,
        pltpu.SemaphoreType.DMA,
    ],
)
def _sc_msg(xws_hbm, src_hbm, dst_hbm, out_hbm, acc_sh, is0, id0, is1, id1,
            r0, r1, r2, r3, g0, g1, g2, g3, s0, s1, s2, s3,
            i0, i1, i2, i3):
    cid = lax.axis_index("c")
    sid = lax.axis_index("s")
    wid = cid * NS + sid
    rows = (r0, r1, r2, r3)
    gsem = (g0, g1, g2, g3)
    ssem = (s0, s1, s2, s3)

    def _zrow(i, _):
        def _z16(j, _):
            r0[i, pl.ds(j * 16, 16)] = jnp.zeros((16,), jnp.float32)
            return 0
        lax.fori_loop(0, D // 16, _z16, 0)
        return 0

    lax.fori_loop(0, B, _zrow, 0)

    def _zcpy(k, _):
        pltpu.sync_copy(r0, acc_sh.at[pl.ds(sid * SEG + k * B, B), :])
        return 0

    lax.fori_loop(0, SEG // B, _zcpy, 0)
    pltpu.sync_copy(r0.at[pl.ds(0, SEG - (SEG // B) * B), :],
                    acc_sh.at[pl.ds(sid * SEG + (SEG // B) * B,
                                    SEG - (SEG // B) * B), :])
    plsc.subcore_barrier()

    pltpu.async_copy(src_hbm.at[wid, pl.ds(0, BLK)], is0, i0)
    pltpu.async_copy(dst_hbm.at[wid, pl.ds(0, BLK)], id0, i1)
    pltpu.async_copy(src_hbm.at[wid, pl.ds(BLK, BLK)], is1, i2)
    pltpu.async_copy(dst_hbm.at[wid, pl.ds(BLK, BLK)], id1, i3)
    pltpu.make_async_copy(src_hbm.at[wid, pl.ds(0, BLK)], is0, i0).wait()
    pltpu.make_async_copy(dst_hbm.at[wid, pl.ds(0, BLK)], id0, i1).wait()
    for _p in range(NBUF):
        pltpu.async_copy(xws_hbm.at[is0.at[_p]], rows[_p], gsem[_p])

    def _run_block(b, idx_s, idx_d, sem_is, sem_id, n_s, n_d, sem_nis,
                   sem_nid, has_next):

        def _grp(u, _):
            j0 = NBUF * u
            for p in range(NBUF):
                j = j0 + p
                q = (p + NBUF - 1) % NBUF
                pltpu.make_async_copy(
                    xws_hbm.at[idx_s.at[j]], rows[p], gsem[p]).wait()
                pltpu.async_copy(rows[p], acc_sh.at[idx_d.at[j]], ssem[p],
                                 add=True)

                @pl.when((j >= 1) & (j + NBUF - 1 < BLK))
                def _(j=j, q=q):
                    pltpu.make_async_copy(
                        rows[q], acc_sh.at[idx_d.at[j - 1]], ssem[q]).wait()
                    pltpu.async_copy(
                        xws_hbm.at[idx_s.at[j + NBUF - 1]], rows[q], gsem[q])
            return 0

        lax.fori_loop(0, BLK // NBUF, _grp, 0)
        if has_next:
            pltpu.make_async_copy(
                src_hbm.at[wid, pl.ds((b + 1) * BLK, BLK)], n_s,
                sem_nis).wait()
            pltpu.make_async_copy(
                dst_hbm.at[wid, pl.ds((b + 1) * BLK, BLK)], n_d,
                sem_nid).wait()
        for i in range(NBUF):
            pltpu.make_async_copy(
                rows[i], acc_sh.at[idx_d.at[BLK - NBUF + i]],
                ssem[i]).wait()
            if has_next:
                pltpu.async_copy(xws_hbm.at[n_s.at[i]], rows[i], gsem[i])

        @pl.when(b + 2 < NBLK)
        def _():
            pltpu.async_copy(
                src_hbm.at[wid, pl.ds((b + 2) * BLK, BLK)], idx_s, sem_is)
            pltpu.async_copy(
                dst_hbm.at[wid, pl.ds((b + 2) * BLK, BLK)], idx_d, sem_id)

    assert NBLK % 2 == 1

    def _bpair(v, _):
        _run_block(2 * v, is0, id0, i0, i1, is1, id1, i2, i3, True)
        _run_block(2 * v + 1, is1, id1, i2, i3, is0, id0, i0, i1, True)
        return 0

    lax.fori_loop(0, NBLK // 2, _bpair, 0)
    _run_block(NBLK - 1, is0, id0, i0, i1, is1, id1, i2, i3, False)
    plsc.subcore_barrier()

    pltpu.sync_copy(acc_sh.at[pl.ds(sid * SEG, SEG), :],
                    out_hbm.at[cid, pl.ds(sid * SEG, SEG), :])


def _tc_first_body(x_ref, w_ref, deg_ref, out_ref):
    dis = lax.rsqrt(deg_ref[...])
    xw = jnp.dot(x_ref[...], w_ref[...], preferred_element_type=jnp.float32)
    out_ref[...] = xw * dis


def _tc_mid_body(acc_ref, xws_ref, deg_ref, w_ref, b_ref, g_ref, be_ref,
                 out_ref):
    dis = lax.rsqrt(deg_ref[...])
    acc = acc_ref[0] + acc_ref[1]
    conv = (acc + xws_ref[...]) * dis + b_ref[...]
    gs = g_ref[...] * lax.rsqrt(jnp.float32(1.0 + BN_EPS))
    h = jnp.maximum(conv * gs + be_ref[...], 0.0)
    xw = jnp.dot(h, w_ref[...], preferred_element_type=jnp.float32)
    out_ref[...] = xw * dis


def _tc_last_body(acc_ref, xws_ref, deg_ref, b_ref, g_ref, be_ref, out_ref):
    dis = lax.rsqrt(deg_ref[...])
    acc = acc_ref[0] + acc_ref[1]
    conv = (acc + xws_ref[...]) * dis + b_ref[...]
    gs = g_ref[...] * lax.rsqrt(jnp.float32(1.0 + BN_EPS))
    out_ref[...] = jnp.maximum(conv * gs + be_ref[...], 0.0)


BR = 2000
_row = pl.BlockSpec((BR, D), lambda i: (i, 0))
_deg_bs = pl.BlockSpec((BR, 1), lambda i: (i, 0))
_acc_bs = pl.BlockSpec((2, BR, D), lambda i: (0, i, 0))
_w_bs = pl.BlockSpec((D, D), lambda i: (0, 0))
_vec_bs = pl.BlockSpec((1, D), lambda i: (0, 0))


def _tc_first(x, w, deg):
    return pl.pallas_call(
        _tc_first_body,
        grid=(N // BR,),
        in_specs=[_row, _w_bs, _deg_bs],
        out_specs=_row,
        out_shape=jax.ShapeDtypeStruct((N, D), jnp.float32),
    )(x, w, deg)


def _tc_mid(acc, xws, deg, w, b, g, be):
    return pl.pallas_call(
        _tc_mid_body,
        grid=(N // BR,),
        in_specs=[_acc_bs, _row, _deg_bs, _w_bs, _vec_bs, _vec_bs, _vec_bs],
        out_specs=_row,
        out_shape=jax.ShapeDtypeStruct((N, D), jnp.float32),
    )(acc, xws, deg, w, b, g, be)


def _tc_last(acc, xws, deg, b, g, be):
    return pl.pallas_call(
        _tc_last_body,
        grid=(N // BR,),
        in_specs=[_acc_bs, _row, _deg_bs, _vec_bs, _vec_bs, _vec_bs],
        out_specs=_row,
        out_shape=jax.ShapeDtypeStruct((N, D), jnp.float32),
    )(acc, xws, deg, b, g, be)


@jax.jit
def kernel(x, edge_index, W1, b1, g1, be1, W2, b2, g2, be2, Wf, bf, gf, bef):
    src = edge_index[0]
    dst = edge_index[1]
    pad = EPAD - E
    ar = jnp.arange(pad, dtype=jnp.int32)
    src_p = jnp.concatenate([src, (ar * 37) % N]).reshape(NW, CH, B)
    dst_p = jnp.concatenate([dst, N + (ar % 96)]).reshape(NW, CH, B)

    degp = _sc_degree(dst_p)
    deg = (degp[:N] + degp[NPAD:NPAD + N] + 1.0)[:, None]

    b1r, g1r, be1r = b1[None, :], g1[None, :], be1[None, :]
    b2r, g2r, be2r = b2[None, :], g2[None, :], be2[None, :]
    bfr, gfr, befr = bf[None, :], gf[None, :], bef[None, :]

    xws1 = _tc_first(x, W1, deg)
    acc1 = _sc_msg(xws1, src_p, dst_p)
    xws2 = _tc_mid(acc1, xws1, deg, W2, b1r, g1r, be1r)
    acc2 = _sc_msg(xws2, src_p, dst_p)
    xws3 = _tc_mid(acc2, xws2, deg, Wf, b2r, g2r, be2r)
    acc3 = _sc_msg(xws3, src_p, dst_p)
    return _tc_last(acc3, xws3, deg, bfr, gfr, befr)


# speedup vs baseline: 1.0866x; 1.0102x over previous
import functools

import jax
import jax.numpy as jnp
from jax import lax
from jax.experimental import pallas as pl
from jax.experimental.pallas import tpu as pltpu
from jax.experimental.pallas import tpu_sc as plsc

N = 10000
E = 320000
D = 128
BN_EPS = 1e-5

NC = 2
NS = 16
NW = NC * NS
B = 64
NBUF = 4
BLK = 32
NBLK = 5
CH = BLK * NBLK
EPAD = NW * CH * B
NPAD = 10112
SEG = NPAD // NS

_mesh = plsc.VectorSubcoreMesh(core_axis_name="c", subcore_axis_name="s")


@functools.partial(
    pl.kernel,
    out_type=jax.ShapeDtypeStruct((NC * NPAD,), jnp.float32),
    mesh=_mesh,
    scratch_types=[
        pltpu.VMEM_SHARED((NPAD,), jnp.float32),
        pltpu.VMEM((CH, B), jnp.int32),
        pltpu.VMEM((B,), jnp.float32),
        pltpu.VMEM((640,), jnp.float32),
        pltpu.SemaphoreType.DMA,
    ],
)
def _sc_degree(dst_hbm, out_hbm, deg_sh, idx_d, ones_v, zeros_v, sem_d):
    cid = lax.axis_index("c")
    sid = lax.axis_index("s")
    wid = cid * NS + sid

    def _fill_ones(i, _):
        ones_v[pl.ds(i * 16, 16)] = jnp.full((16,), 1.0, jnp.float32)
        return 0

    def _fill_zeros(i, _):
        zeros_v[pl.ds(i * 16, 16)] = jnp.zeros((16,), jnp.float32)
        return 0

    lax.fori_loop(0, B // 16, _fill_ones, 0)
    lax.fori_loop(0, 640 // 16, _fill_zeros, 0)

    pltpu.sync_copy(dst_hbm.at[wid], idx_d)
    pltpu.sync_copy(zeros_v.at[pl.ds(0, SEG)],
                    deg_sh.at[pl.ds(sid * SEG, SEG)])
    plsc.subcore_barrier()

    def _fire(j, _):
        pltpu.async_copy(ones_v, deg_sh.at[idx_d.at[j]], sem_d, add=True)
        return 0

    def _drain(j, _):
        pltpu.make_async_copy(ones_v, deg_sh.at[idx_d.at[j]], sem_d).wait()
        return 0

    lax.fori_loop(0, CH, _fire, 0)
    lax.fori_loop(0, CH, _drain, 0)
    plsc.subcore_barrier()
    pltpu.sync_copy(deg_sh.at[pl.ds(sid * SEG, SEG)],
                    zeros_v.at[pl.ds(0, SEG)])
    pltpu.sync_copy(zeros_v.at[pl.ds(0, SEG)],
                    out_hbm.at[pl.ds(cid * NPAD + sid * SEG, SEG)])


@functools.partial(
    pl.kernel,
    out_type=jax.ShapeDtypeStruct((NC, NPAD, D), jnp.float32),
    mesh=_mesh,
    scratch_types=[
        pltpu.VMEM_SHARED((NPAD, D), jnp.float32),
        pltpu.VMEM((BLK, B), jnp.int32),
        pltpu.VMEM((BLK, B), jnp.int32),
        pltpu.VMEM((BLK, B), jnp.int32),
        pltpu.VMEM((BLK, B), jnp.int32),
        pltpu.VMEM((B, D), jnp.float32),
        pltpu.VMEM((B, D), jnp.float32),
        pltpu.VMEM((B, D), jnp.float32),
        pltpu.VMEM((B, D), jnp.float32),
        pltpu.SemaphoreType.DMA,
        pltpu.SemaphoreType.DMA,
        pltpu.SemaphoreType.DMA,
        pltpu.SemaphoreType.DMA,
        pltpu.SemaphoreType.DMA,
        pltpu.SemaphoreType.DMA,
        pltpu.SemaphoreType.DMA,
        pltpu.SemaphoreType.DMA,
        pltpu.SemaphoreType.DMA,
        pltpu.SemaphoreType.DMA,
        pltpu.SemaphoreType.DMA,
        pltpu.SemaphoreType.DMA,
    ],
)
def _sc_msg(xws_hbm, src_hbm, dst_hbm, out_hbm, acc_sh, is0, id0, is1, id1,
            r0, r1, r2, r3, g0, g1, g2, g3, s0, s1, s2, s3,
            i0, i1, i2, i3):
    cid = lax.axis_index("c")
    sid = lax.axis_index("s")
    wid = cid * NS + sid
    rows = (r0, r1, r2, r3)
    gsem = (g0, g1, g2, g3)
    ssem = (s0, s1, s2, s3)

    pltpu.async_copy(src_hbm.at[wid, pl.ds(0, BLK)], is0, i0)
    pltpu.async_copy(dst_hbm.at[wid, pl.ds(0, BLK)], id0, i1)
    pltpu.async_copy(src_hbm.at[wid, pl.ds(BLK, BLK)], is1, i2)
    pltpu.async_copy(dst_hbm.at[wid, pl.ds(BLK, BLK)], id1, i3)

    def _zrow(i, _):
        def _z16(j, _):
            r0[i, pl.ds(j * 16, 16)] = jnp.zeros((16,), jnp.float32)
            return 0
        lax.fori_loop(0, D // 16, _z16, 0)
        return 0

    lax.fori_loop(0, B, _zrow, 0)

    ZT = SEG - (SEG // B) * B
    for k in range(SEG // B):
        pltpu.async_copy(r0, acc_sh.at[pl.ds(sid * SEG + k * B, B), :],
                         ssem[k % NBUF])
    pltpu.async_copy(r0.at[pl.ds(0, ZT), :],
                     acc_sh.at[pl.ds(sid * SEG + (SEG // B) * B, ZT), :],
                     ssem[(SEG // B) % NBUF])
    for k in range(SEG // B):
        pltpu.make_async_copy(
            r0, acc_sh.at[pl.ds(sid * SEG + k * B, B), :],
            ssem[k % NBUF]).wait()
    pltpu.make_async_copy(
        r0.at[pl.ds(0, ZT), :],
        acc_sh.at[pl.ds(sid * SEG + (SEG // B) * B, ZT), :],
        ssem[(SEG // B) % NBUF]).wait()
    plsc.subcore_barrier()

    pltpu.make_async_copy(src_hbm.at[wid, pl.ds(0, BLK)], is0, i0).wait()
    pltpu.make_async_copy(dst_hbm.at[wid, pl.ds(0, BLK)], id0, i1).wait()
    for _p in range(NBUF):
        pltpu.async_copy(xws_hbm.at[is0.at[_p]], rows[_p], gsem[_p])

    def _run_block(b, idx_s, idx_d, sem_is, sem_id, n_s, n_d, sem_nis,
                   sem_nid, has_next):

        def _grp(u, _):
            j0 = NBUF * u
            for p in range(NBUF):
                j = j0 + p
                q = (p + NBUF - 1) % NBUF
                pltpu.make_async_copy(
                    xws_hbm.at[idx_s.at[j]], rows[p], gsem[p]).wait()
                pltpu.async_copy(rows[p], acc_sh.at[idx_d.at[j]], ssem[p],
                                 add=True)

                @pl.when((j >= 1) & (j + NBUF - 1 < BLK))
                def _(j=j, q=q):
                    pltpu.make_async_copy(
                        rows[q], acc_sh.at[idx_d.at[j - 1]], ssem[q]).wait()
                    pltpu.async_copy(
                        xws_hbm.at[idx_s.at[j + NBUF - 1]], rows[q], gsem[q])
            return 0

        lax.fori_loop(0, BLK // NBUF, _grp, 0)
        if has_next:
            pltpu.make_async_copy(
                src_hbm.at[wid, pl.ds((b + 1) * BLK, BLK)], n_s,
                sem_nis).wait()
            pltpu.make_async_copy(
                dst_hbm.at[wid, pl.ds((b + 1) * BLK, BLK)], n_d,
                sem_nid).wait()
        for i in range(NBUF):
            pltpu.make_async_copy(
                rows[i], acc_sh.at[idx_d.at[BLK - NBUF + i]],
                ssem[i]).wait()
            if has_next:
                pltpu.async_copy(xws_hbm.at[n_s.at[i]], rows[i], gsem[i])

        @pl.when(b + 2 < NBLK)
        def _():
            pltpu.async_copy(
                src_hbm.at[wid, pl.ds((b + 2) * BLK, BLK)], idx_s, sem_is)
            pltpu.async_copy(
                dst_hbm.at[wid, pl.ds((b + 2) * BLK, BLK)], idx_d, sem_id)

    assert NBLK % 2 == 1

    def _bpair(v, _):
        _run_block(2 * v, is0, id0, i0, i1, is1, id1, i2, i3, True)
        _run_block(2 * v + 1, is1, id1, i2, i3, is0, id0, i0, i1, True)
        return 0

    lax.fori_loop(0, NBLK // 2, _bpair, 0)
    _run_block(NBLK - 1, is0, id0, i0, i1, is1, id1, i2, i3, False)
    plsc.subcore_barrier()

    pltpu.sync_copy(acc_sh.at[pl.ds(sid * SEG, SEG), :],
                    out_hbm.at[cid, pl.ds(sid * SEG, SEG), :])


def _tc_first_body(x_ref, w_ref, deg_ref, out_ref):
    dis = lax.rsqrt(deg_ref[...])
    xw = jnp.dot(x_ref[...], w_ref[...], preferred_element_type=jnp.float32)
    out_ref[...] = xw * dis


def _tc_mid_body(acc_ref, xws_ref, deg_ref, w_ref, b_ref, g_ref, be_ref,
                 out_ref):
    dis = lax.rsqrt(deg_ref[...])
    acc = acc_ref[0] + acc_ref[1]
    conv = (acc + xws_ref[...]) * dis + b_ref[...]
    gs = g_ref[...] * lax.rsqrt(jnp.float32(1.0 + BN_EPS))
    h = jnp.maximum(conv * gs + be_ref[...], 0.0)
    xw = jnp.dot(h, w_ref[...], preferred_element_type=jnp.float32)
    out_ref[...] = xw * dis


def _tc_last_body(acc_ref, xws_ref, deg_ref, b_ref, g_ref, be_ref, out_ref):
    dis = lax.rsqrt(deg_ref[...])
    acc = acc_ref[0] + acc_ref[1]
    conv = (acc + xws_ref[...]) * dis + b_ref[...]
    gs = g_ref[...] * lax.rsqrt(jnp.float32(1.0 + BN_EPS))
    out_ref[...] = jnp.maximum(conv * gs + be_ref[...], 0.0)


BR = 2000
_row = pl.BlockSpec((BR, D), lambda i: (i, 0))
_deg_bs = pl.BlockSpec((BR, 1), lambda i: (i, 0))
_acc_bs = pl.BlockSpec((2, BR, D), lambda i: (0, i, 0))
_w_bs = pl.BlockSpec((D, D), lambda i: (0, 0))
_vec_bs = pl.BlockSpec((1, D), lambda i: (0, 0))


def _tc_first(x, w, deg):
    return pl.pallas_call(
        _tc_first_body,
        grid=(N // BR,),
        in_specs=[_row, _w_bs, _deg_bs],
        out_specs=_row,
        out_shape=jax.ShapeDtypeStruct((N, D), jnp.float32),
    )(x, w, deg)


def _tc_mid(acc, xws, deg, w, b, g, be):
    return pl.pallas_call(
        _tc_mid_body,
        grid=(N // BR,),
        in_specs=[_acc_bs, _row, _deg_bs, _w_bs, _vec_bs, _vec_bs, _vec_bs],
        out_specs=_row,
        out_shape=jax.ShapeDtypeStruct((N, D), jnp.float32),
    )(acc, xws, deg, w, b, g, be)


def _tc_last(acc, xws, deg, b, g, be):
    return pl.pallas_call(
        _tc_last_body,
        grid=(N // BR,),
        in_specs=[_acc_bs, _row, _deg_bs, _vec_bs, _vec_bs, _vec_bs],
        out_specs=_row,
        out_shape=jax.ShapeDtypeStruct((N, D), jnp.float32),
    )(acc, xws, deg, b, g, be)


@jax.jit
def kernel(x, edge_index, W1, b1, g1, be1, W2, b2, g2, be2, Wf, bf, gf, bef):
    src = edge_index[0]
    dst = edge_index[1]
    pad = EPAD - E
    ar = jnp.arange(pad, dtype=jnp.int32)
    src_p = jnp.concatenate([src, (ar * 37) % N]).reshape(NW, CH, B)
    dst_p = jnp.concatenate([dst, N + (ar % 96)]).reshape(NW, CH, B)

    degp = _sc_degree(dst_p)
    deg = (degp[:N] + degp[NPAD:NPAD + N] + 1.0)[:, None]

    b1r, g1r, be1r = b1[None, :], g1[None, :], be1[None, :]
    b2r, g2r, be2r = b2[None, :], g2[None, :], be2[None, :]
    bfr, gfr, befr = bf[None, :], gf[None, :], bef[None, :]

    xws1 = _tc_first(x, W1, deg)
    acc1 = _sc_msg(xws1, src_p, dst_p)
    xws2 = _tc_mid(acc1, xws1, deg, W2, b1r, g1r, be1r)
    acc2 = _sc_msg(xws2, src_p, dst_p)
    xws3 = _tc_mid(acc2, xws2, deg, Wf, b2r, g2r, be2r)
    acc3 = _sc_msg(xws3, src_p, dst_p)
    return _tc_last(acc3, xws3, deg, bfr, gfr, befr)
